# Initial kernel scaffold; baseline (speedup 1.0000x reference)
import functools

import jax
import jax.numpy as jnp
from jax import lax
from jax.experimental import pallas as pl
from jax.experimental.pallas import tpu as pltpu
from jax.experimental.pallas import tpu_sc as plsc

N = 10000
D = 128
E = 320000
NC = 2
NS = 16
NW = NC * NS
EPT = E // NW
K = 80
CHUNKS = EPT // K
RPT = N // NS
CW = 16


def _sc_body(x_hbm, row_hbm, col_hbm, acc_out, cnt_out,
             row_v, col_v, rows_v, ones_v, zrow_v, zcnt_v,
             acc_sh, cnt_sh, gsem):
    cid = lax.axis_index("c")
    sid = lax.axis_index("s")
    wid = sid * NC + cid

    def fill_ones(i, _):
        ones_v[i] = jnp.ones((CW,), jnp.float32)
        return 0
    lax.fori_loop(0, K, fill_ones, 0)

    def zero_rows(t, _):
        i = t // (D // 16)
        k = t % (D // 16)
        zrow_v[i, pl.ds(k * 16, 16)] = jnp.zeros((16,), jnp.float32)
        return 0
    lax.fori_loop(0, (RPT // 5) * (D // 16), zero_rows, 0)

    def zero_cnt(i, _):
        zcnt_v[i] = jnp.zeros((CW,), jnp.float32)
        return 0
    lax.fori_loop(0, RPT, zero_cnt, 0)

    rbase = sid * RPT
    for r in range(5):
        pltpu.sync_copy(zrow_v, acc_sh.at[pl.ds(rbase + r * (RPT // 5),
                                                RPT // 5)])
    pltpu.sync_copy(zcnt_v, cnt_sh.at[pl.ds(rbase, RPT)])
    plsc.subcore_barrier()

    ebase = wid * EPT

    def body(j, _):
        base = ebase + j * K
        pltpu.sync_copy(row_hbm.at[pl.ds(base, K)], row_v)
        pltpu.sync_copy(col_hbm.at[pl.ds(base, K)], col_v)
        pltpu.async_copy(x_hbm.at[row_v], rows_v, gsem).wait()
        pltpu.sync_copy(rows_v, acc_sh.at[col_v], add=True)
        pltpu.sync_copy(ones_v, cnt_sh.at[col_v], add=True)
        return 0
    lax.fori_loop(0, CHUNKS, body, 0)

    plsc.subcore_barrier()

    pltpu.sync_copy(acc_sh.at[pl.ds(rbase, RPT)],
                    acc_out.at[cid, pl.ds(rbase, RPT)])
    pltpu.sync_copy(cnt_sh.at[pl.ds(rbase, RPT)],
                    cnt_out.at[cid, pl.ds(rbase, RPT)])


_sc_kernel = functools.partial(
    pl.kernel,
    out_type=(
        jax.ShapeDtypeStruct((NC, N, D), jnp.float32),
        jax.ShapeDtypeStruct((NC, N, CW), jnp.float32),
    ),
    mesh=plsc.VectorSubcoreMesh(core_axis_name="c", subcore_axis_name="s"),
    scratch_types=[
        pltpu.VMEM((K,), jnp.int32),
        pltpu.VMEM((K,), jnp.int32),
        pltpu.VMEM((K, D), jnp.float32),
        pltpu.VMEM((K, CW), jnp.float32),
        pltpu.VMEM((RPT // 5, D), jnp.float32),
        pltpu.VMEM((RPT, CW), jnp.float32),
        pltpu.VMEM_SHARED((N, D), jnp.float32),
        pltpu.VMEM_SHARED((N, CW), jnp.float32),
        pltpu.SemaphoreType.DMA,
    ],
)(_sc_body)


BN = 1000


def _tc_body(x_ref, acc_ref, cnt_ref, out_ref):
    cnt = jnp.sum(cnt_ref[...], axis=(0, 2))
    s = acc_ref[0] + acc_ref[1]
    inv = 1.0 / jnp.maximum(cnt, 1.0)
    out_ref[:, :D] = s * inv[:, None]
    mask = jnp.where(cnt > 0.0, 1.0, 0.0)
    out_ref[:, D:] = x_ref[...] * mask[:, None]


_tc_finalize = pl.pallas_call(
    _tc_body,
    grid=(N // BN,),
    in_specs=[
        pl.BlockSpec((BN, D), lambda i: (i, 0)),
        pl.BlockSpec((NC, BN, D), lambda i: (0, i, 0)),
        pl.BlockSpec((NC, BN, CW), lambda i: (0, i, 0)),
    ],
    out_specs=pl.BlockSpec((BN, 2 * D), lambda i: (i, 0)),
    out_shape=jax.ShapeDtypeStruct((N, 2 * D), jnp.float32),
)


@jax.jit
def kernel(x, es):
    col = es[0].astype(jnp.int32)
    row = es[1].astype(jnp.int32)
    acc, cnt = _sc_kernel(x, row, col)
    return _tc_finalize(x, acc, cnt)


# speedup vs baseline: 9.9403x; 9.9403x over previous
import functools

import jax
import jax.numpy as jnp
from jax import lax
from jax.experimental import pallas as pl
from jax.experimental.pallas import tpu as pltpu
from jax.experimental.pallas import tpu_sc as plsc

N = 10000
D = 128
E = 320000
NC = 2
NS = 16
NW = NC * NS
EPT = E // NW
K = 80
CHUNKS = EPT // K
DR = 1000
NDT = N // DR
ZR = 40
ZC = 200
CW = 16


def _sc_body(x_hbm, row_hbm, col_hbm, acc_out, cnt_out,
             row_v, col_v, rows_v, ones_v, zrow_v, zcnt_v,
             acc_sh, cnt_sh, gsem):
    cid = lax.axis_index("c")
    sid = lax.axis_index("s")
    wid = sid * NC + cid

    def fill_ones(i, _):
        ones_v[i] = jnp.ones((CW,), jnp.float32)
        return 0
    lax.fori_loop(0, K, fill_ones, 0)

    def zero_rows(t, _):
        i = t // (D // 16)
        k = t % (D // 16)
        zrow_v[i, pl.ds(k * 16, 16)] = jnp.zeros((16,), jnp.float32)
        return 0
    lax.fori_loop(0, ZR * (D // 16), zero_rows, 0)

    def zero_cnt(i, _):
        zcnt_v[i] = jnp.zeros((CW,), jnp.float32)
        return 0
    lax.fori_loop(0, ZC, zero_cnt, 0)

    rbase = sid * DR

    @pl.when(sid < NDT)
    def _init():
        for r in range(DR // ZR):
            pltpu.sync_copy(zrow_v, acc_sh.at[pl.ds(rbase + r * ZR, ZR)])
        for r in range(DR // ZC):
            pltpu.sync_copy(zcnt_v, cnt_sh.at[pl.ds(rbase + r * ZC, ZC)])

    plsc.subcore_barrier()

    ebase = wid * EPT

    def body(j, _):
        base = ebase + j * K
        pltpu.sync_copy(row_hbm.at[pl.ds(base, K)], row_v)
        pltpu.sync_copy(col_hbm.at[pl.ds(base, K)], col_v)
        pltpu.async_copy(x_hbm.at[row_v], rows_v, gsem).wait()
        pltpu.sync_copy(rows_v, acc_sh.at[col_v], add=True)
        pltpu.sync_copy(ones_v, cnt_sh.at[col_v], add=True)
        return 0
    lax.fori_loop(0, CHUNKS, body, 0)

    plsc.subcore_barrier()

    @pl.when(sid < NDT)
    def _drain():
        pltpu.sync_copy(acc_sh.at[pl.ds(rbase, DR)],
                        acc_out.at[cid, pl.ds(rbase, DR)])
        pltpu.sync_copy(cnt_sh.at[pl.ds(rbase, DR)],
                        cnt_out.at[cid, pl.ds(rbase, DR)])


_sc_kernel = functools.partial(
    pl.kernel,
    out_type=(
        jax.ShapeDtypeStruct((NC, N, D), jnp.float32),
        jax.ShapeDtypeStruct((NC, N, CW), jnp.float32),
    ),
    mesh=plsc.VectorSubcoreMesh(core_axis_name="c", subcore_axis_name="s"),
    scratch_types=[
        pltpu.VMEM((K,), jnp.int32),
        pltpu.VMEM((K,), jnp.int32),
        pltpu.VMEM((K, D), jnp.float32),
        pltpu.VMEM((K, CW), jnp.float32),
        pltpu.VMEM((ZR, D), jnp.float32),
        pltpu.VMEM((ZC, CW), jnp.float32),
        pltpu.VMEM_SHARED((N, D), jnp.float32),
        pltpu.VMEM_SHARED((N, CW), jnp.float32),
        pltpu.SemaphoreType.DMA,
    ],
    compiler_params=pltpu.CompilerParams(use_tc_tiling_on_sc=False),
)(_sc_body)


BN = 1000


def _tc_body(x_ref, acc_ref, cnt_ref, out_ref):
    cnt = jnp.sum(cnt_ref[...], axis=(0, 2)) * (1.0 / CW)
    s = acc_ref[0] + acc_ref[1]
    inv = 1.0 / jnp.maximum(cnt, 1.0)
    out_ref[:, :D] = s * inv[:, None]
    mask = jnp.where(cnt > 0.0, 1.0, 0.0)
    out_ref[:, D:] = x_ref[...] * mask[:, None]


_tc_finalize = pl.pallas_call(
    _tc_body,
    grid=(N // BN,),
    in_specs=[
        pl.BlockSpec((BN, D), lambda i: (i, 0)),
        pl.BlockSpec((NC, BN, D), lambda i: (0, i, 0)),
        pl.BlockSpec((NC, BN, CW), lambda i: (0, i, 0)),
    ],
    out_specs=pl.BlockSpec((BN, 2 * D), lambda i: (i, 0)),
    out_shape=jax.ShapeDtypeStruct((N, 2 * D), jnp.float32),
)


@jax.jit
def kernel(x, es):
    col = es[0].astype(jnp.int32)
    row = es[1].astype(jnp.int32)
    acc, cnt = _sc_kernel(x, row, col)
    return _tc_finalize(x, acc, cnt)


# speedup vs baseline: 17.5462x; 1.7652x over previous
import functools

import jax
import jax.numpy as jnp
from jax import lax
from jax.experimental import pallas as pl
from jax.experimental.pallas import tpu as pltpu
from jax.experimental.pallas import tpu_sc as plsc

N = 10000
D = 128
E = 320000
NC = 2
NS = 16
NW = NC * NS
EPT = E // NW
K = 80
CHUNKS = EPT // K
DR = 1000
NDT = N // DR
CW = 8


def _sc_body(x_hbm, row_hbm, col_hbm, zrow_hbm, zcnt_hbm,
             acc_out, cnt_out,
             row_idx, col_idx, rows0, rows1, ones_v,
             acc_sh, cnt_sh, gsem0, gsem1, ssem0, ssem1):
    cid = lax.axis_index("c")
    sid = lax.axis_index("s")
    wid = sid * NC + cid

    pltpu.sync_copy(row_hbm.at[pl.ds(wid * CHUNKS, CHUNKS)], row_idx)
    pltpu.sync_copy(col_hbm.at[pl.ds(wid * CHUNKS, CHUNKS)], col_idx)

    def fill_ones(i, _):
        ones_v[i] = jnp.ones((CW,), jnp.float32)
        return 0
    lax.fori_loop(0, K, fill_ones, 0)

    rbase = sid * DR

    @pl.when(sid < NDT)
    def _init():
        pltpu.sync_copy(zrow_hbm, acc_sh.at[pl.ds(rbase, DR)])
        pltpu.sync_copy(zcnt_hbm, cnt_sh.at[pl.ds(rbase, DR)])

    plsc.subcore_barrier()

    def g_start(j, buf, sem):
        return pltpu.async_copy(x_hbm.at[row_idx.at[j]], buf, sem)

    def g_wait(buf, sem):
        pltpu.make_async_copy(x_hbm.at[row_idx.at[0]], buf, sem).wait()

    def s_start(j, buf, sem):
        pltpu.async_copy(buf, acc_sh.at[col_idx.at[j]], sem, add=True)
        pltpu.async_copy(ones_v, cnt_sh.at[col_idx.at[j]], sem, add=True)

    def s_wait(buf, sem):
        pltpu.make_async_copy(buf, acc_sh.at[col_idx.at[0]], sem).wait()
        pltpu.make_async_copy(ones_v, cnt_sh.at[col_idx.at[0]], sem).wait()

    g_start(0, rows0, gsem0)
    g_start(1, rows1, gsem1)

    def body(t, _):
        j = 2 * t
        g_wait(rows0, gsem0)
        s_start(j, rows0, ssem0)
        g_wait(rows1, gsem1)
        s_start(j + 1, rows1, ssem1)
        s_wait(rows0, ssem0)
        g_start(j + 2, rows0, gsem0)
        s_wait(rows1, ssem1)

        @pl.when(j + 3 < CHUNKS)
        def _():
            g_start(j + 3, rows1, gsem1)
        return 0
    lax.fori_loop(0, (CHUNKS - 1) // 2, body, 0)

    g_wait(rows0, gsem0)
    s_start(CHUNKS - 1, rows0, ssem0)
    s_wait(rows0, ssem0)

    plsc.subcore_barrier()

    @pl.when(sid < NDT)
    def _drain():
        pltpu.sync_copy(acc_sh.at[pl.ds(rbase, DR)],
                        acc_out.at[cid, pl.ds(rbase, DR)])
        pltpu.sync_copy(cnt_sh.at[pl.ds(rbase, DR)],
                        cnt_out.at[cid, pl.ds(rbase, DR)])


_sc_kernel = functools.partial(
    pl.kernel,
    out_type=(
        jax.ShapeDtypeStruct((NC, N, D), jnp.float32),
        jax.ShapeDtypeStruct((NC, N, CW), jnp.float32),
    ),
    mesh=plsc.VectorSubcoreMesh(core_axis_name="c", subcore_axis_name="s"),
    scratch_types=[
        pltpu.VMEM((CHUNKS, K), jnp.int32),
        pltpu.VMEM((CHUNKS, K), jnp.int32),
        pltpu.VMEM((K, D), jnp.float32),
        pltpu.VMEM((K, D), jnp.float32),
        pltpu.VMEM((K, CW), jnp.float32),
        pltpu.VMEM_SHARED((N, D), jnp.float32),
        pltpu.VMEM_SHARED((N, CW), jnp.float32),
        pltpu.SemaphoreType.DMA,
        pltpu.SemaphoreType.DMA,
        pltpu.SemaphoreType.DMA,
        pltpu.SemaphoreType.DMA,
    ],
    compiler_params=pltpu.CompilerParams(use_tc_tiling_on_sc=False),
)(_sc_body)


BN = 1000


def _tc_body(x_ref, acc_ref, cnt_ref, out_ref):
    cnt = jnp.sum(cnt_ref[...], axis=(0, 2)) * (1.0 / CW)
    s = acc_ref[0] + acc_ref[1]
    inv = 1.0 / jnp.maximum(cnt, 1.0)
    out_ref[:, :D] = s * inv[:, None]
    mask = jnp.where(cnt > 0.0, 1.0, 0.0)
    out_ref[:, D:] = x_ref[...] * mask[:, None]


_tc_finalize = pl.pallas_call(
    _tc_body,
    grid=(N // BN,),
    in_specs=[
        pl.BlockSpec((BN, D), lambda i: (i, 0)),
        pl.BlockSpec((NC, BN, D), lambda i: (0, i, 0)),
        pl.BlockSpec((NC, BN, CW), lambda i: (0, i, 0)),
    ],
    out_specs=pl.BlockSpec((BN, 2 * D), lambda i: (i, 0)),
    out_shape=jax.ShapeDtypeStruct((N, 2 * D), jnp.float32),
)


@jax.jit
def kernel(x, es):
    col = es[0].astype(jnp.int32).reshape(E // K, K)
    row = es[1].astype(jnp.int32).reshape(E // K, K)
    zrow = jnp.zeros((DR, D), jnp.float32)
    zcnt = jnp.zeros((DR, CW), jnp.float32)
    acc, cnt = _sc_kernel(x, row, col, zrow, zcnt)
    return _tc_finalize(x, acc, cnt)
